# baseline (speedup 1.0000x reference)
import jax
import jax.numpy as jnp
from jax import lax
from jax.experimental import pallas as pl
from jax.experimental.pallas import tpu as pltpu
from jax.experimental.pallas import tpu_sc as plsc

N = 20000
ROWS = 160
LANES = 128
NP = ROWS * LANES
MAX_OUT = 300
IOU_THRESH = 0.5
NEG = -1e9
HUGE = 3e38

NT = 16
SLOTS = NP // NT
CHUNKS = SLOTS // 16


def _prep_body(x1_ref, y1_ref, x2_ref, y2_ref, lab_ref,
               bx1_ref, by1_ref, bx2_ref, by2_ref, ar_ref):
    X1 = x1_ref[:]
    Y1 = y1_ref[:]
    X2 = x2_ref[:]
    Y2 = y2_ref[:]
    LAB = lab_ref[:]
    mc = jnp.max(jnp.maximum(jnp.maximum(X1, X2), jnp.maximum(Y1, Y2)))
    off = LAB * (mc + 1.0)
    BX1 = X1 + off
    BY1 = Y1 + off
    BX2 = X2 + off
    BY2 = Y2 + off
    bx1_ref[:] = BX1
    by1_ref[:] = BY1
    bx2_ref[:] = BX2
    by2_ref[:] = BY2
    ar_ref[:] = (BX2 - BX1) * (BY2 - BY1)


def _sc_body(bx1_h, by1_h, bx2_h, by2_h, ar_h, rem_h,
             ox1_h, oy1_h, ox2_h, oy2_h, out_h, ex_h,
             bx1, by1, bx2, by2, ar, rem, ox1, oy1, ox2, oy2,
             recv, recs_v, outbuf):
    cid = lax.axis_index("c")
    sid = lax.axis_index("s")

    @pl.when(cid == 0)
    def _():
        base = sid * SLOTS
        for h, v in ((bx1_h, bx1), (by1_h, by1), (bx2_h, bx2), (by2_h, by2),
                     (ar_h, ar), (rem_h, rem), (ox1_h, ox1), (oy1_h, oy1),
                     (ox2_h, ox2), (oy2_h, oy2)):
            pltpu.sync_copy(h.at[pl.ds(base, SLOTS)], v.at[pl.ds(0, SLOTS)])

        li = lax.iota(jnp.int32, 16)
        lif = li.astype(jnp.float32)
        basef = jnp.full((16,), base, jnp.int32).astype(jnp.float32)
        zero16 = jnp.zeros((16,), jnp.float32)
        f05 = jnp.float32(IOU_THRESH)
        fhuge = jnp.float32(HUGE)

        def ins2(m1, i1, m2, i2, v, iv):
            b1 = (v > m1) | ((v == m1) & (iv < i1))
            b2 = (v > m2) | ((v == m2) & (iv < i2))
            nm1 = jnp.where(b1, v, m1)
            ni1 = jnp.where(b1, iv, i1)
            nm2 = jnp.where(b1, m1, jnp.where(b2, v, m2))
            ni2 = jnp.where(b1, i1, jnp.where(b2, iv, i2))
            return nm1, ni1, nm2, ni2

        def round_body(par32m, carry):
            (t,
             a1s, b1s, c1s, d1s, e1s, th1,
             a2s, b2s, c2s, d2s, e2s, th2) = carry
            active = t < MAX_OUT
            pbx1 = jnp.full((16,), a1s, jnp.float32)
            pby1 = jnp.full((16,), b1s, jnp.float32)
            pbx2 = jnp.full((16,), c1s, jnp.float32)
            pby2 = jnp.full((16,), d1s, jnp.float32)
            par1 = jnp.full((16,), e1s, jnp.float32)
            thr1 = jnp.full((16,), th1, jnp.float32)
            qbx1 = jnp.full((16,), a2s, jnp.float32)
            qby1 = jnp.full((16,), b2s, jnp.float32)
            qbx2 = jnp.full((16,), c2s, jnp.float32)
            qby2 = jnp.full((16,), d2s, jnp.float32)
            qar = jnp.full((16,), e2s, jnp.float32)
            thr2 = jnp.full((16,), th2, jnp.float32)

            def chunk(i, c):
                best, bidx, sec, sidx, idxv = c
                s = i * 16
                r = rem[pl.ds(s, 16)]
                a1 = bx1[pl.ds(s, 16)]
                b1 = by1[pl.ds(s, 16)]
                a2 = bx2[pl.ds(s, 16)]
                b2 = by2[pl.ds(s, 16)]
                aa = ar[pl.ds(s, 16)]

                def iou_of(wx1, wy1, wx2, wy2, war):
                    xx1 = jnp.maximum(wx1, a1)
                    yy1 = jnp.maximum(wy1, b1)
                    xx2 = jnp.minimum(wx2, a2)
                    yy2 = jnp.minimum(wy2, b2)
                    inter = jnp.maximum(xx2 - xx1, 0.0) * \
                        jnp.maximum(yy2 - yy1, 0.0)
                    union = war + aa - inter
                    return inter / jnp.maximum(union, 1e-9)

                i1v = iou_of(pbx1, pby1, pbx2, pby2, par1)
                rn = jnp.where(i1v > thr1, jnp.float32(NEG), r)
                i2v = iou_of(qbx1, qby1, qbx2, qby2, qar)
                rn = jnp.where(i2v > thr2, jnp.float32(NEG), rn)
                rem[pl.ds(s, 16)] = rn
                gt1 = rn > best
                gt2 = rn > sec
                sec = jnp.where(gt1, best, jnp.where(gt2, rn, sec))
                sidx = jnp.where(gt1, bidx, jnp.where(gt2, idxv, sidx))
                best = jnp.where(gt1, rn, best)
                bidx = jnp.where(gt1, idxv, bidx)
                return best, bidx, sec, sidx, idxv + 16.0

            init = (jnp.full((16,), jnp.float32(-2e9)), basef + lif,
                    jnp.full((16,), jnp.float32(-2e9)), basef + lif,
                    basef + lif)
            nch = jnp.where(active, jnp.int32(CHUNKS), jnp.int32(0))
            best, bidx, sec, sidx, _ = lax.fori_loop(0, nch, chunk, init)

            m1 = best[0]
            i1 = bidx[0]
            m2 = sec[0]
            i2 = sidx[0]
            for l in range(1, 16):
                m1, i1, m2, i2 = ins2(m1, i1, m2, i2, best[l], bidx[l])
            for l in range(16):
                m1, i1, m2, i2 = ins2(m1, i1, m2, i2, sec[l], sidx[l])

            def mkrec(lm, lidx):
                loff = lidx.astype(jnp.int32) - base

                def ext(ref):
                    return ref[pl.ds(loff, 16)][0]

                fields = [lm, lidx, ext(bx1), ext(by1), ext(bx2), ext(by2),
                          ext(ar), ext(ox1), ext(oy1), ext(ox2), ext(oy2)]
                rec = zero16
                for j, f in enumerate(fields):
                    rec = jnp.where(li == j,
                                    jnp.full((16,), f, jnp.float32), rec)
                return rec

            rec1 = mkrec(m1, i1)
            rec2 = mkrec(m2, i2)

            @pl.when(active)
            def _():
                recv[:] = rec1
                pltpu.sync_copy(recv, ex_h.at[par32m + sid])
                recv[:] = rec2
                pltpu.sync_copy(recv, ex_h.at[par32m + 16 + sid])
                plsc.subcore_barrier()
                pltpu.sync_copy(ex_h.at[pl.ds(par32m, 32)], recs_v)

            row0 = recs_v[0, :]
            g1 = row0[0]
            gi1 = row0[1]
            r1 = jnp.int32(0)
            g2 = jnp.float32(-3e9)
            gi2 = jnp.float32(0.0)
            r2 = jnp.int32(0)
            for rr in range(1, 32):
                rowr = recs_v[rr, :]
                v = rowr[0]
                iv = rowr[1]
                b1 = (v > g1) | ((v == g1) & (iv < gi1))
                b2 = (v > g2) | ((v == g2) & (iv < gi2))
                ng1 = jnp.where(b1, v, g1)
                ngi1 = jnp.where(b1, iv, gi1)
                nr1 = jnp.where(b1, jnp.int32(rr), r1)
                g2 = jnp.where(b1, g1, jnp.where(b2, v, g2))
                gi2 = jnp.where(b1, gi1, jnp.where(b2, iv, gi2))
                r2 = jnp.where(b1, r1, jnp.where(b2, jnp.int32(rr), r2))
                g1, gi1, r1 = ng1, ngi1, nr1
            w1 = recs_v[r1, :]
            w2 = recs_v[r2, :]

            valid1 = g1 > -1e8
            vf1 = jnp.where(valid1, jnp.float32(1.0), jnp.float32(0.0))
            valid2 = g2 > -1e8
            vf2 = jnp.where(valid2, jnp.float32(1.0), jnp.float32(0.0))

            wx1 = jnp.full((16,), w1[2], jnp.float32)
            wy1 = jnp.full((16,), w1[3], jnp.float32)
            wx2 = jnp.full((16,), w1[4], jnp.float32)
            wy2 = jnp.full((16,), w1[5], jnp.float32)
            wa = jnp.full((16,), w1[6], jnp.float32)
            ux1 = jnp.full((16,), w2[2], jnp.float32)
            uy1 = jnp.full((16,), w2[3], jnp.float32)
            ux2 = jnp.full((16,), w2[4], jnp.float32)
            uy2 = jnp.full((16,), w2[5], jnp.float32)
            ua = jnp.full((16,), w2[6], jnp.float32)
            xx1 = jnp.maximum(wx1, ux1)
            yy1 = jnp.maximum(wy1, uy1)
            xx2 = jnp.minimum(wx2, ux2)
            yy2 = jnp.minimum(wy2, uy2)
            inter = jnp.maximum(xx2 - xx1, 0.0) * jnp.maximum(yy2 - yy1, 0.0)
            union = wa + ua - inter
            recv[:] = inter / jnp.maximum(union, 1e-9)
            iou12 = recv[:][0]
            supp2 = (iou12 > IOU_THRESH) & valid1
            consume2 = (~supp2) & (t + 1 < MAX_OUT) & active

            @pl.when((sid == 0) & active)
            def _():
                orow = zero16
                for j, f in enumerate([w1[7], w1[8], w1[9], w1[10], g1]):
                    orow = jnp.where(
                        li == j, jnp.full((16,), f * vf1, jnp.float32), orow)
                outbuf[t, :] = orow

            @pl.when((sid == 0) & consume2)
            def _():
                orow = zero16
                for j, f in enumerate([w2[7], w2[8], w2[9], w2[10], g2]):
                    orow = jnp.where(
                        li == j, jnp.full((16,), f * vf2, jnp.float32), orow)
                outbuf[t + 1, :] = orow

            nth1 = jnp.where(valid1, f05, fhuge)
            nth2 = jnp.where(consume2 & valid2, f05, fhuge)
            tn = jnp.where(active, t + 1 + consume2.astype(jnp.int32), t)

            def sel(new, old):
                return jnp.where(active, new, old)

            return (tn,
                    sel(w1[2], a1s), sel(w1[3], b1s), sel(w1[4], c1s),
                    sel(w1[5], d1s), sel(w1[6], e1s), sel(nth1, th1),
                    sel(w2[2], a2s), sel(w2[3], b2s), sel(w2[4], c2s),
                    sel(w2[5], d2s), sel(w2[6], e2s), sel(nth2, th2))

        def pair(_, carry):
            carry = round_body(0, carry)
            carry = round_body(32, carry)
            return carry

        z = jnp.float32(0.0)
        lax.fori_loop(
            0, MAX_OUT // 2, pair,
            (jnp.int32(0),
             z, z, z, z, z, fhuge,
             z, z, z, z, z, fhuge))

        @pl.when(sid == 0)
        def _():
            pltpu.sync_copy(outbuf, out_h)


def kernel(boxes, scores, labels):
    def pad2d(v, fill):
        v = jnp.concatenate(
            [v, jnp.full((NP - N,), fill, jnp.float32)])
        return v.reshape(ROWS, LANES)

    x1 = pad2d(boxes[:, 0], 0.0)
    y1 = pad2d(boxes[:, 1], 0.0)
    x2 = pad2d(boxes[:, 2], 0.0)
    y2 = pad2d(boxes[:, 3], 0.0)
    sc = pad2d(scores, NEG)
    lab = pad2d(labels.astype(jnp.float32), 0.0)

    plane = jax.ShapeDtypeStruct((ROWS, LANES), jnp.float32)
    bx1, by1, bx2, by2, ar = pl.pallas_call(
        _prep_body,
        out_shape=[plane] * 5,
    )(x1, y1, x2, y2, lab)

    mesh = plsc.VectorSubcoreMesh(core_axis_name="c", subcore_axis_name="s",
                                  num_cores=2, num_subcores=16)
    sc_kernel = pl.kernel(
        _sc_body,
        out_type=[jax.ShapeDtypeStruct((MAX_OUT, 16), jnp.float32),
                  jax.ShapeDtypeStruct((64, 16), jnp.float32)],
        mesh=mesh,
        scratch_types=(
            [pltpu.VMEM((SLOTS + 16,), jnp.float32) for _ in range(10)]
            + [pltpu.VMEM((16,), jnp.float32),
               pltpu.VMEM((32, 16), jnp.float32),
               pltpu.VMEM((MAX_OUT, 16), jnp.float32)]),
    )
    out, _ = sc_kernel(bx1.reshape(NP), by1.reshape(NP), bx2.reshape(NP),
                       by2.reshape(NP), ar.reshape(NP), sc.reshape(NP),
                       x1.reshape(NP), y1.reshape(NP), x2.reshape(NP),
                       y2.reshape(NP))
    return out[:, :5]


# speedup vs baseline: 1.1353x; 1.1353x over previous
import jax
import jax.numpy as jnp
from jax import lax
from jax.experimental import pallas as pl
from jax.experimental.pallas import tpu as pltpu
from jax.experimental.pallas import tpu_sc as plsc

N = 20000
ROWS = 160
LANES = 128
NP = ROWS * LANES
MAX_OUT = 300
IOU_THRESH = 0.5
NEG = -1e9
HUGE = 3e38

NT = 16
SLOTS = NP // NT
CHUNKS = SLOTS // 16


def _prep_body(x1_ref, y1_ref, x2_ref, y2_ref, lab_ref,
               bx1_ref, by1_ref, bx2_ref, by2_ref, ar_ref):
    X1 = x1_ref[:]
    Y1 = y1_ref[:]
    X2 = x2_ref[:]
    Y2 = y2_ref[:]
    LAB = lab_ref[:]
    mc = jnp.max(jnp.maximum(jnp.maximum(X1, X2), jnp.maximum(Y1, Y2)))
    off = LAB * (mc + 1.0)
    BX1 = X1 + off
    BY1 = Y1 + off
    BX2 = X2 + off
    BY2 = Y2 + off
    bx1_ref[:] = BX1
    by1_ref[:] = BY1
    bx2_ref[:] = BX2
    by2_ref[:] = BY2
    ar_ref[:] = (BX2 - BX1) * (BY2 - BY1)


def _sc_body(bx1_h, by1_h, bx2_h, by2_h, ar_h, rem_h,
             ox1_h, oy1_h, ox2_h, oy2_h, out_h, ex_h,
             bx1, by1, bx2, by2, ar, rem, ox1, oy1, ox2, oy2,
             recv, recv2, recs_v, outbuf, sem1, sem2):
    cid = lax.axis_index("c")
    sid = lax.axis_index("s")

    @pl.when(cid == 0)
    def _():
        base = sid * SLOTS
        for h, v in ((bx1_h, bx1), (by1_h, by1), (bx2_h, bx2), (by2_h, by2),
                     (ar_h, ar), (rem_h, rem), (ox1_h, ox1), (oy1_h, oy1),
                     (ox2_h, ox2), (oy2_h, oy2)):
            pltpu.sync_copy(h.at[pl.ds(base, SLOTS)], v.at[pl.ds(0, SLOTS)])

        li = lax.iota(jnp.int32, 16)
        lif = li.astype(jnp.float32)
        basef = jnp.full((16,), base, jnp.int32).astype(jnp.float32)
        zero16 = jnp.zeros((16,), jnp.float32)
        f05 = jnp.float32(IOU_THRESH)
        fhuge = jnp.float32(HUGE)

        def ins2(m1, i1, m2, i2, v, iv):
            b1 = (v > m1) | ((v == m1) & (iv < i1))
            b2 = (v > m2) | ((v == m2) & (iv < i2))
            nm1 = jnp.where(b1, v, m1)
            ni1 = jnp.where(b1, iv, i1)
            nm2 = jnp.where(b1, m1, jnp.where(b2, v, m2))
            ni2 = jnp.where(b1, i1, jnp.where(b2, iv, i2))
            return nm1, ni1, nm2, ni2

        def round_body(par32m, carry):
            (t,
             a1s, b1s, c1s, d1s, e1s, th1,
             a2s, b2s, c2s, d2s, e2s, th2) = carry
            active = t < MAX_OUT
            pbx1 = jnp.full((16,), a1s, jnp.float32)
            pby1 = jnp.full((16,), b1s, jnp.float32)
            pbx2 = jnp.full((16,), c1s, jnp.float32)
            pby2 = jnp.full((16,), d1s, jnp.float32)
            par1 = jnp.full((16,), e1s, jnp.float32)
            thr1 = jnp.full((16,), th1, jnp.float32)
            qbx1 = jnp.full((16,), a2s, jnp.float32)
            qby1 = jnp.full((16,), b2s, jnp.float32)
            qbx2 = jnp.full((16,), c2s, jnp.float32)
            qby2 = jnp.full((16,), d2s, jnp.float32)
            qar = jnp.full((16,), e2s, jnp.float32)
            thr2 = jnp.full((16,), th2, jnp.float32)

            def chunk(i, c):
                best, bidx, sec, sidx, idxv = c
                s = i * 16
                r = rem[pl.ds(s, 16)]
                a1 = bx1[pl.ds(s, 16)]
                b1 = by1[pl.ds(s, 16)]
                a2 = bx2[pl.ds(s, 16)]
                b2 = by2[pl.ds(s, 16)]
                aa = ar[pl.ds(s, 16)]

                def iou_of(wx1, wy1, wx2, wy2, war):
                    xx1 = jnp.maximum(wx1, a1)
                    yy1 = jnp.maximum(wy1, b1)
                    xx2 = jnp.minimum(wx2, a2)
                    yy2 = jnp.minimum(wy2, b2)
                    inter = jnp.maximum(xx2 - xx1, 0.0) * \
                        jnp.maximum(yy2 - yy1, 0.0)
                    union = war + aa - inter
                    return inter / jnp.maximum(union, 1e-9)

                i1v = iou_of(pbx1, pby1, pbx2, pby2, par1)
                rn = jnp.where(i1v > thr1, jnp.float32(NEG), r)
                i2v = iou_of(qbx1, qby1, qbx2, qby2, qar)
                rn = jnp.where(i2v > thr2, jnp.float32(NEG), rn)
                rem[pl.ds(s, 16)] = rn
                gt1 = rn > best
                gt2 = rn > sec
                sec = jnp.where(gt1, best, jnp.where(gt2, rn, sec))
                sidx = jnp.where(gt1, bidx, jnp.where(gt2, idxv, sidx))
                best = jnp.where(gt1, rn, best)
                bidx = jnp.where(gt1, idxv, bidx)
                return best, bidx, sec, sidx, idxv + 16.0

            init = (jnp.full((16,), jnp.float32(-2e9)), basef + lif,
                    jnp.full((16,), jnp.float32(-2e9)), basef + lif,
                    basef + lif)
            nch = jnp.where(active, jnp.int32(CHUNKS), jnp.int32(0))
            best, bidx, sec, sidx, _ = lax.fori_loop(0, nch, chunk, init)

            m1 = best[0]
            i1 = bidx[0]
            m2 = sec[0]
            i2 = sidx[0]
            for l in range(1, 16):
                m1, i1, m2, i2 = ins2(m1, i1, m2, i2, best[l], bidx[l])
            for l in range(16):
                m1, i1, m2, i2 = ins2(m1, i1, m2, i2, sec[l], sidx[l])

            def mkrec(lm, lidx):
                loff = lidx.astype(jnp.int32) - base

                def ext(ref):
                    return ref[pl.ds(loff, 16)][0]

                fields = [lm, lidx, ext(bx1), ext(by1), ext(bx2), ext(by2),
                          ext(ar), ext(ox1), ext(oy1), ext(ox2), ext(oy2)]
                rec = zero16
                for j, f in enumerate(fields):
                    rec = jnp.where(li == j,
                                    jnp.full((16,), f, jnp.float32), rec)
                return rec

            rec1 = mkrec(m1, i1)
            rec2 = mkrec(m2, i2)

            @pl.when(active)
            def _():
                recv[:] = rec1
                recv2[:] = rec2
                c1 = pltpu.async_copy(recv, ex_h.at[par32m + sid], sem1)
                c2 = pltpu.async_copy(recv2, ex_h.at[par32m + 16 + sid], sem2)
                c1.wait()
                c2.wait()
                plsc.subcore_barrier()
                pltpu.sync_copy(ex_h.at[pl.ds(par32m, 32)], recs_v)

            def scan_row(rr, c):
                g1, gi1, r1, g2, gi2, r2 = c
                rowr = recs_v[rr, :]
                v = rowr[0]
                iv = rowr[1]
                b1 = (v > g1) | ((v == g1) & (iv < gi1))
                b2 = (v > g2) | ((v == g2) & (iv < gi2))
                ng1 = jnp.where(b1, v, g1)
                ngi1 = jnp.where(b1, iv, gi1)
                nr1 = jnp.where(b1, rr, r1)
                g2 = jnp.where(b1, g1, jnp.where(b2, v, g2))
                gi2 = jnp.where(b1, gi1, jnp.where(b2, iv, gi2))
                r2 = jnp.where(b1, r1, jnp.where(b2, rr, r2))
                return ng1, ngi1, nr1, g2, gi2, r2

            nscan = jnp.where(active, jnp.int32(32), jnp.int32(0))
            g1, gi1, r1, g2, gi2, r2 = lax.fori_loop(
                0, nscan, scan_row,
                (jnp.float32(-4e9), jnp.float32(0.0), jnp.int32(0),
                 jnp.float32(-4e9), jnp.float32(0.0), jnp.int32(0)))
            w1 = recs_v[r1, :]
            w2 = recs_v[r2, :]

            valid1 = g1 > -1e8
            vf1 = jnp.where(valid1, jnp.float32(1.0), jnp.float32(0.0))
            valid2 = g2 > -1e8
            vf2 = jnp.where(valid2, jnp.float32(1.0), jnp.float32(0.0))

            wx1 = jnp.full((16,), w1[2], jnp.float32)
            wy1 = jnp.full((16,), w1[3], jnp.float32)
            wx2 = jnp.full((16,), w1[4], jnp.float32)
            wy2 = jnp.full((16,), w1[5], jnp.float32)
            wa = jnp.full((16,), w1[6], jnp.float32)
            ux1 = jnp.full((16,), w2[2], jnp.float32)
            uy1 = jnp.full((16,), w2[3], jnp.float32)
            ux2 = jnp.full((16,), w2[4], jnp.float32)
            uy2 = jnp.full((16,), w2[5], jnp.float32)
            ua = jnp.full((16,), w2[6], jnp.float32)
            xx1 = jnp.maximum(wx1, ux1)
            yy1 = jnp.maximum(wy1, uy1)
            xx2 = jnp.minimum(wx2, ux2)
            yy2 = jnp.minimum(wy2, uy2)
            inter = jnp.maximum(xx2 - xx1, 0.0) * jnp.maximum(yy2 - yy1, 0.0)
            union = wa + ua - inter
            recv[:] = inter / jnp.maximum(union, 1e-9)
            iou12 = recv[:][0]
            supp2 = (iou12 > IOU_THRESH) & valid1
            consume2 = (~supp2) & (t + 1 < MAX_OUT) & active

            @pl.when((sid == 0) & active)
            def _():
                orow = zero16
                for j, f in enumerate([w1[7], w1[8], w1[9], w1[10], g1]):
                    orow = jnp.where(
                        li == j, jnp.full((16,), f * vf1, jnp.float32), orow)
                outbuf[t, :] = orow

            @pl.when((sid == 0) & consume2)
            def _():
                orow = zero16
                for j, f in enumerate([w2[7], w2[8], w2[9], w2[10], g2]):
                    orow = jnp.where(
                        li == j, jnp.full((16,), f * vf2, jnp.float32), orow)
                outbuf[t + 1, :] = orow

            nth1 = jnp.where(valid1, f05, fhuge)
            nth2 = jnp.where(consume2 & valid2, f05, fhuge)
            tn = jnp.where(active, t + 1 + consume2.astype(jnp.int32), t)

            def sel(new, old):
                return jnp.where(active, new, old)

            return (tn,
                    sel(w1[2], a1s), sel(w1[3], b1s), sel(w1[4], c1s),
                    sel(w1[5], d1s), sel(w1[6], e1s), sel(nth1, th1),
                    sel(w2[2], a2s), sel(w2[3], b2s), sel(w2[4], c2s),
                    sel(w2[5], d2s), sel(w2[6], e2s), sel(nth2, th2))

        def pair(_, carry):
            carry = round_body(0, carry)
            carry = round_body(32, carry)
            return carry

        z = jnp.float32(0.0)
        lax.fori_loop(
            0, MAX_OUT // 2, pair,
            (jnp.int32(0),
             z, z, z, z, z, fhuge,
             z, z, z, z, z, fhuge))

        @pl.when(sid == 0)
        def _():
            pltpu.sync_copy(outbuf, out_h)


def kernel(boxes, scores, labels):
    def pad2d(v, fill):
        v = jnp.concatenate(
            [v, jnp.full((NP - N,), fill, jnp.float32)])
        return v.reshape(ROWS, LANES)

    x1 = pad2d(boxes[:, 0], 0.0)
    y1 = pad2d(boxes[:, 1], 0.0)
    x2 = pad2d(boxes[:, 2], 0.0)
    y2 = pad2d(boxes[:, 3], 0.0)
    sc = pad2d(scores, NEG)
    lab = pad2d(labels.astype(jnp.float32), 0.0)

    plane = jax.ShapeDtypeStruct((ROWS, LANES), jnp.float32)
    bx1, by1, bx2, by2, ar = pl.pallas_call(
        _prep_body,
        out_shape=[plane] * 5,
    )(x1, y1, x2, y2, lab)

    mesh = plsc.VectorSubcoreMesh(core_axis_name="c", subcore_axis_name="s",
                                  num_cores=2, num_subcores=16)
    sc_kernel = pl.kernel(
        _sc_body,
        out_type=[jax.ShapeDtypeStruct((MAX_OUT, 16), jnp.float32),
                  jax.ShapeDtypeStruct((64, 16), jnp.float32)],
        mesh=mesh,
        scratch_types=(
            [pltpu.VMEM((SLOTS + 16,), jnp.float32) for _ in range(10)]
            + [pltpu.VMEM((16,), jnp.float32),
               pltpu.VMEM((16,), jnp.float32),
               pltpu.VMEM((32, 16), jnp.float32),
               pltpu.VMEM((MAX_OUT, 16), jnp.float32),
               pltpu.SemaphoreType.DMA,
               pltpu.SemaphoreType.DMA]),
    )
    out, _ = sc_kernel(bx1.reshape(NP), by1.reshape(NP), bx2.reshape(NP),
                       by2.reshape(NP), ar.reshape(NP), sc.reshape(NP),
                       x1.reshape(NP), y1.reshape(NP), x2.reshape(NP),
                       y2.reshape(NP))
    return out[:, :5]


# speedup vs baseline: 1.1471x; 1.0103x over previous
import jax
import jax.numpy as jnp
from jax import lax
from jax.experimental import pallas as pl
from jax.experimental.pallas import tpu as pltpu
from jax.experimental.pallas import tpu_sc as plsc

N = 20000
ROWS = 160
LANES = 128
NP = ROWS * LANES
MAX_OUT = 300
IOU_THRESH = 0.5
NEG = -1e9
HUGE = 3e38

NT = 16
SLOTS = NP // NT
CHUNKS = SLOTS // 16


def _prep_body(x1_ref, y1_ref, x2_ref, y2_ref, lab_ref,
               bx1_ref, by1_ref, bx2_ref, by2_ref, ar_ref):
    X1 = x1_ref[:]
    Y1 = y1_ref[:]
    X2 = x2_ref[:]
    Y2 = y2_ref[:]
    LAB = lab_ref[:]
    mc = jnp.max(jnp.maximum(jnp.maximum(X1, X2), jnp.maximum(Y1, Y2)))
    off = LAB * (mc + 1.0)
    BX1 = X1 + off
    BY1 = Y1 + off
    BX2 = X2 + off
    BY2 = Y2 + off
    bx1_ref[:] = BX1
    by1_ref[:] = BY1
    bx2_ref[:] = BX2
    by2_ref[:] = BY2
    ar_ref[:] = (BX2 - BX1) * (BY2 - BY1)


def _sc_body(bx1_h, by1_h, bx2_h, by2_h, ar_h, rem_h,
             ox1_h, oy1_h, ox2_h, oy2_h, out_h, ex_h,
             bx1, by1, bx2, by2, ar, rem, ox1, oy1, ox2, oy2,
             recv, recv2, recs_v, outbuf, sem1, sem2):
    cid = lax.axis_index("c")
    sid = lax.axis_index("s")

    @pl.when(cid == 0)
    def _():
        base = sid * SLOTS
        for h, v in ((bx1_h, bx1), (by1_h, by1), (bx2_h, bx2), (by2_h, by2),
                     (ar_h, ar), (rem_h, rem), (ox1_h, ox1), (oy1_h, oy1),
                     (ox2_h, ox2), (oy2_h, oy2)):
            pltpu.sync_copy(h.at[pl.ds(base, SLOTS)], v.at[pl.ds(0, SLOTS)])

        li = lax.iota(jnp.int32, 16)
        lif = li.astype(jnp.float32)
        basef = jnp.full((16,), base, jnp.int32).astype(jnp.float32)
        zero16 = jnp.zeros((16,), jnp.float32)
        f05 = jnp.float32(IOU_THRESH)
        fhuge = jnp.float32(HUGE)

        def ins2(m1, i1, m2, i2, v, iv):
            b1 = (v > m1) | ((v == m1) & (iv < i1))
            b2 = (v > m2) | ((v == m2) & (iv < i2))
            nm1 = jnp.where(b1, v, m1)
            ni1 = jnp.where(b1, iv, i1)
            nm2 = jnp.where(b1, m1, jnp.where(b2, v, m2))
            ni2 = jnp.where(b1, i1, jnp.where(b2, iv, i2))
            return nm1, ni1, nm2, ni2

        def round_body(par32m, carry):
            (t,
             a1s, b1s, c1s, d1s, e1s, th1,
             a2s, b2s, c2s, d2s, e2s, th2) = carry
            active = t < MAX_OUT
            pbx1 = jnp.full((16,), a1s, jnp.float32)
            pby1 = jnp.full((16,), b1s, jnp.float32)
            pbx2 = jnp.full((16,), c1s, jnp.float32)
            pby2 = jnp.full((16,), d1s, jnp.float32)
            par1 = jnp.full((16,), e1s, jnp.float32)
            thr1 = jnp.full((16,), th1, jnp.float32)
            qbx1 = jnp.full((16,), a2s, jnp.float32)
            qby1 = jnp.full((16,), b2s, jnp.float32)
            qbx2 = jnp.full((16,), c2s, jnp.float32)
            qby2 = jnp.full((16,), d2s, jnp.float32)
            qar = jnp.full((16,), e2s, jnp.float32)
            thr2 = jnp.full((16,), th2, jnp.float32)

            def chunk(i, c):
                best, bidx, sec, sidx, idxv = c
                s = i * 16
                r = rem[pl.ds(s, 16)]
                a1 = bx1[pl.ds(s, 16)]
                b1 = by1[pl.ds(s, 16)]
                a2 = bx2[pl.ds(s, 16)]
                b2 = by2[pl.ds(s, 16)]
                aa = ar[pl.ds(s, 16)]

                def iou_of(wx1, wy1, wx2, wy2, war):
                    xx1 = jnp.maximum(wx1, a1)
                    yy1 = jnp.maximum(wy1, b1)
                    xx2 = jnp.minimum(wx2, a2)
                    yy2 = jnp.minimum(wy2, b2)
                    inter = jnp.maximum(xx2 - xx1, 0.0) * \
                        jnp.maximum(yy2 - yy1, 0.0)
                    union = war + aa - inter
                    return inter / jnp.maximum(union, 1e-9)

                i1v = iou_of(pbx1, pby1, pbx2, pby2, par1)
                rn = jnp.where(i1v > thr1, jnp.float32(NEG), r)
                i2v = iou_of(qbx1, qby1, qbx2, qby2, qar)
                rn = jnp.where(i2v > thr2, jnp.float32(NEG), rn)
                rem[pl.ds(s, 16)] = rn
                gt1 = rn > best
                gt2 = rn > sec
                sec = jnp.where(gt1, best, jnp.where(gt2, rn, sec))
                sidx = jnp.where(gt1, bidx, jnp.where(gt2, idxv, sidx))
                best = jnp.where(gt1, rn, best)
                bidx = jnp.where(gt1, idxv, bidx)
                return best, bidx, sec, sidx, idxv + 16.0

            init = (jnp.full((16,), jnp.float32(-2e9)), basef + lif,
                    jnp.full((16,), jnp.float32(-2e9)), basef + lif,
                    basef + lif)
            nch = jnp.where(active, jnp.int32(CHUNKS), jnp.int32(0))
            best, bidx, sec, sidx, _ = lax.fori_loop(0, nch, chunk, init)

            def mkrec(lm, lidx):
                loff = lidx.astype(jnp.int32) - base

                def ext(ref):
                    return ref[pl.ds(loff, 16)][0]

                fields = [lm, lidx, ext(bx1), ext(by1), ext(bx2), ext(by2),
                          ext(ar), ext(ox1), ext(oy1), ext(ox2), ext(oy2)]
                rec = zero16
                for j, f in enumerate(fields):
                    rec = jnp.where(li == j,
                                    jnp.full((16,), f, jnp.float32), rec)
                return rec

            @pl.when(active)
            def _():
                m1 = best[0]
                i1 = bidx[0]
                m2 = sec[0]
                i2 = sidx[0]
                for l in range(1, 16):
                    m1, i1, m2, i2 = ins2(m1, i1, m2, i2, best[l], bidx[l])
                for l in range(16):
                    m1, i1, m2, i2 = ins2(m1, i1, m2, i2, sec[l], sidx[l])
                recv[:] = mkrec(m1, i1)
                recv2[:] = mkrec(m2, i2)
                c1 = pltpu.async_copy(recv, ex_h.at[par32m + sid], sem1)
                c2 = pltpu.async_copy(recv2, ex_h.at[par32m + 16 + sid], sem2)
                c1.wait()
                c2.wait()
                plsc.subcore_barrier()
                pltpu.sync_copy(ex_h.at[pl.ds(par32m, 32)], recs_v)

            def scan_row(rr, c):
                g1, gi1, r1, g2, gi2, r2 = c
                rowr = recs_v[rr, :]
                v = rowr[0]
                iv = rowr[1]
                b1 = (v > g1) | ((v == g1) & (iv < gi1))
                b2 = (v > g2) | ((v == g2) & (iv < gi2))
                ng1 = jnp.where(b1, v, g1)
                ngi1 = jnp.where(b1, iv, gi1)
                nr1 = jnp.where(b1, rr, r1)
                g2 = jnp.where(b1, g1, jnp.where(b2, v, g2))
                gi2 = jnp.where(b1, gi1, jnp.where(b2, iv, gi2))
                r2 = jnp.where(b1, r1, jnp.where(b2, rr, r2))
                return ng1, ngi1, nr1, g2, gi2, r2

            nscan = jnp.where(active, jnp.int32(32), jnp.int32(0))
            g1, gi1, r1, g2, gi2, r2 = lax.fori_loop(
                0, nscan, scan_row,
                (jnp.float32(-4e9), jnp.float32(0.0), jnp.int32(0),
                 jnp.float32(-4e9), jnp.float32(0.0), jnp.int32(0)))
            w1 = recs_v[r1, :]
            w2 = recs_v[r2, :]

            valid1 = g1 > -1e8
            vf1 = jnp.where(valid1, jnp.float32(1.0), jnp.float32(0.0))
            valid2 = g2 > -1e8
            vf2 = jnp.where(valid2, jnp.float32(1.0), jnp.float32(0.0))

            wx1 = jnp.full((16,), w1[2], jnp.float32)
            wy1 = jnp.full((16,), w1[3], jnp.float32)
            wx2 = jnp.full((16,), w1[4], jnp.float32)
            wy2 = jnp.full((16,), w1[5], jnp.float32)
            wa = jnp.full((16,), w1[6], jnp.float32)
            ux1 = jnp.full((16,), w2[2], jnp.float32)
            uy1 = jnp.full((16,), w2[3], jnp.float32)
            ux2 = jnp.full((16,), w2[4], jnp.float32)
            uy2 = jnp.full((16,), w2[5], jnp.float32)
            ua = jnp.full((16,), w2[6], jnp.float32)
            xx1 = jnp.maximum(wx1, ux1)
            yy1 = jnp.maximum(wy1, uy1)
            xx2 = jnp.minimum(wx2, ux2)
            yy2 = jnp.minimum(wy2, uy2)
            inter = jnp.maximum(xx2 - xx1, 0.0) * jnp.maximum(yy2 - yy1, 0.0)
            union = wa + ua - inter
            recv[:] = inter / jnp.maximum(union, 1e-9)
            iou12 = recv[:][0]
            supp2 = (iou12 > IOU_THRESH) & valid1
            consume2 = (~supp2) & (t + 1 < MAX_OUT) & active

            @pl.when((sid == 0) & active)
            def _():
                orow = zero16
                for j, f in enumerate([w1[7], w1[8], w1[9], w1[10], g1]):
                    orow = jnp.where(
                        li == j, jnp.full((16,), f * vf1, jnp.float32), orow)
                outbuf[t, :] = orow

            @pl.when((sid == 0) & consume2)
            def _():
                orow = zero16
                for j, f in enumerate([w2[7], w2[8], w2[9], w2[10], g2]):
                    orow = jnp.where(
                        li == j, jnp.full((16,), f * vf2, jnp.float32), orow)
                outbuf[t + 1, :] = orow

            nth1 = jnp.where(valid1, f05, fhuge)
            nth2 = jnp.where(consume2 & valid2, f05, fhuge)
            tn = jnp.where(active, t + 1 + consume2.astype(jnp.int32), t)

            def sel(new, old):
                return jnp.where(active, new, old)

            return (tn,
                    sel(w1[2], a1s), sel(w1[3], b1s), sel(w1[4], c1s),
                    sel(w1[5], d1s), sel(w1[6], e1s), sel(nth1, th1),
                    sel(w2[2], a2s), sel(w2[3], b2s), sel(w2[4], c2s),
                    sel(w2[5], d2s), sel(w2[6], e2s), sel(nth2, th2))

        def pair(_, carry):
            carry = round_body(0, carry)
            carry = round_body(32, carry)
            return carry

        z = jnp.float32(0.0)
        lax.fori_loop(
            0, MAX_OUT // 2, pair,
            (jnp.int32(0),
             z, z, z, z, z, fhuge,
             z, z, z, z, z, fhuge))

        @pl.when(sid == 0)
        def _():
            pltpu.sync_copy(outbuf, out_h)


def kernel(boxes, scores, labels):
    def pad2d(v, fill):
        v = jnp.concatenate(
            [v, jnp.full((NP - N,), fill, jnp.float32)])
        return v.reshape(ROWS, LANES)

    x1 = pad2d(boxes[:, 0], 0.0)
    y1 = pad2d(boxes[:, 1], 0.0)
    x2 = pad2d(boxes[:, 2], 0.0)
    y2 = pad2d(boxes[:, 3], 0.0)
    sc = pad2d(scores, NEG)
    lab = pad2d(labels.astype(jnp.float32), 0.0)

    plane = jax.ShapeDtypeStruct((ROWS, LANES), jnp.float32)
    bx1, by1, bx2, by2, ar = pl.pallas_call(
        _prep_body,
        out_shape=[plane] * 5,
    )(x1, y1, x2, y2, lab)

    mesh = plsc.VectorSubcoreMesh(core_axis_name="c", subcore_axis_name="s",
                                  num_cores=2, num_subcores=16)
    sc_kernel = pl.kernel(
        _sc_body,
        out_type=[jax.ShapeDtypeStruct((MAX_OUT, 16), jnp.float32),
                  jax.ShapeDtypeStruct((64, 16), jnp.float32)],
        mesh=mesh,
        scratch_types=(
            [pltpu.VMEM((SLOTS + 16,), jnp.float32) for _ in range(10)]
            + [pltpu.VMEM((16,), jnp.float32),
               pltpu.VMEM((16,), jnp.float32),
               pltpu.VMEM((32, 16), jnp.float32),
               pltpu.VMEM((MAX_OUT, 16), jnp.float32),
               pltpu.SemaphoreType.DMA,
               pltpu.SemaphoreType.DMA]),
    )
    out, _ = sc_kernel(bx1.reshape(NP), by1.reshape(NP), bx2.reshape(NP),
                       by2.reshape(NP), ar.reshape(NP), sc.reshape(NP),
                       x1.reshape(NP), y1.reshape(NP), x2.reshape(NP),
                       y2.reshape(NP))
    return out[:, :5]
